# baseline (speedup 1.0000x reference)
import functools

import jax
import jax.numpy as jnp
from jax import lax
from jax.experimental import pallas as pl
from jax.experimental.pallas import tpu as pltpu
from jax.experimental.pallas import tpu_sc as plsc

N = 10000
E = 320000
NUM_FEATURES = 128
HIDDEN = 64
NUM_CLASSES = 16
K = 10
RANK = 3

NC = 2
NS = 16
NW = NC * NS
CHUNK = 128
UNROLL = 8
CH = 80
EW = CH * CHUNK
EPAD = EW * NW
NPAD = 10112
ZROWS = NPAD // NS
OROWS = ZROWS


def _deg_sc_body(srcd_hbm, ones_hbm, zinit_hbm, degpart_hbm, zsh, didx, ones_v,
                 *ssems):
    c = lax.axis_index("c")
    s = lax.axis_index("s")
    wid = s * NC + c
    pltpu.sync_copy(zinit_hbm, zsh.at[pl.ds(s * ZROWS, ZROWS)])
    pltpu.sync_copy(srcd_hbm.at[wid], didx)
    pltpu.sync_copy(ones_hbm, ones_v)
    plsc.subcore_barrier()

    def body(g, carry):
        j = g * UNROLL
        for u in range(UNROLL):
            pltpu.async_copy(ones_v, zsh.at[didx.at[j + u]], ssems[u],
                             add=True)
        for u in range(UNROLL):
            pltpu.make_async_copy(ones_hbm, ones_v, ssems[u]).wait()
        return carry

    lax.fori_loop(0, CH // UNROLL, body, 0)
    plsc.subcore_barrier()
    pltpu.sync_copy(zsh.at[pl.ds(s * OROWS, OROWS)],
                    degpart_hbm.at[c, pl.ds(s * OROWS, OROWS)])


def _deg_sc(srcd_p, ones_v, zinit):
    mesh = plsc.VectorSubcoreMesh(core_axis_name="c", subcore_axis_name="s")
    fn = pl.kernel(
        _deg_sc_body,
        out_type=jax.ShapeDtypeStruct((NC, NPAD, 8), jnp.float32),
        mesh=mesh,
        scratch_types=(
            [pltpu.VMEM_SHARED((NPAD, 8), jnp.float32),
             pltpu.VMEM((CH, CHUNK), jnp.int32),
             pltpu.VMEM((CHUNK, 8), jnp.float32)]
            + [pltpu.SemaphoreType.DMA] * UNROLL
        ),
        compiler_params=pltpu.CompilerParams(use_tc_tiling_on_sc=False),
    )
    return fn(srcd_p, ones_v, zinit)


def _prop_sc_body(y_hbm, src_hbm, dst_hbm, zinit_hbm, zpart_hbm,
                  zsh, sidx, didx, *bufsems):
    c = lax.axis_index("c")
    s = lax.axis_index("s")
    wid = s * NC + c
    pltpu.sync_copy(zinit_hbm, zsh.at[pl.ds(s * ZROWS, ZROWS)])
    pltpu.sync_copy(src_hbm.at[wid], sidx)
    pltpu.sync_copy(dst_hbm.at[wid], didx)
    plsc.subcore_barrier()

    bufs = bufsems[:UNROLL]
    gsems = bufsems[UNROLL:]

    def group(g, carry):
        j = g * UNROLL
        gds = [pltpu.async_copy(y_hbm.at[sidx.at[j + u]], bufs[u], gsems[u])
               for u in range(UNROLL)]
        for u in range(UNROLL):
            gds[u].wait()
            pltpu.sync_copy(bufs[u], zsh.at[didx.at[j + u]], add=True)
        return carry

    lax.fori_loop(0, CH // UNROLL, group, 0)
    plsc.subcore_barrier()
    pltpu.sync_copy(zsh.at[pl.ds(s * OROWS, OROWS)],
                    zpart_hbm.at[c, pl.ds(s * OROWS, OROWS)])


def _prop_sc(y, srcg_p, dstp_p, zinit):
    mesh = plsc.VectorSubcoreMesh(core_axis_name="c", subcore_axis_name="s")
    fn = pl.kernel(
        _prop_sc_body,
        out_type=jax.ShapeDtypeStruct((NC, NPAD, NUM_CLASSES), jnp.float32),
        mesh=mesh,
        scratch_types=(
            [pltpu.VMEM_SHARED((NPAD, NUM_CLASSES), jnp.float32),
             pltpu.VMEM((CH, CHUNK), jnp.int32),
             pltpu.VMEM((CH, CHUNK), jnp.int32)]
            + [pltpu.VMEM((CHUNK, NUM_CLASSES), jnp.float32)] * UNROLL
            + [pltpu.SemaphoreType.DMA] * UNROLL
        ),
        compiler_params=pltpu.CompilerParams(use_tc_tiling_on_sc=False),
    )
    return fn(y, srcg_p, dstp_p, zinit)


_MLP_ROWS = 2000


def _mlp_body(f_ref, w1_ref, b1_ref, w2_ref, b2_ref, dp_ref,
              x_ref, dis_ref, y_ref):
    h = lax.dot_general(f_ref[...], w1_ref[...], (((1,), (1,)), ((), ())),
                        preferred_element_type=jnp.float32)
    h = jnp.maximum(h + b1_ref[...], 0.0)
    x = lax.dot_general(h, w2_ref[...], (((1,), (1,)), ((), ())),
                        preferred_element_type=jnp.float32)
    x = x + b2_ref[...]
    x_ref[...] = x
    deg = (dp_ref[0] + dp_ref[1])[:, 0:1]
    dis = jnp.where(deg > 0.0, lax.rsqrt(jnp.maximum(deg, 1e-30)), 0.0)
    disb = jnp.broadcast_to(dis, (dis.shape[0], NUM_CLASSES))
    dis_ref[...] = disb
    y_ref[...] = disb * x


def _mlp(feature, W1, b1, W2, b2, degpart):
    grid = N // _MLP_ROWS
    return pl.pallas_call(
        _mlp_body,
        grid=(grid,),
        in_specs=[
            pl.BlockSpec((_MLP_ROWS, NUM_FEATURES), lambda i: (i, 0)),
            pl.BlockSpec((HIDDEN, NUM_FEATURES), lambda i: (0, 0)),
            pl.BlockSpec((1, HIDDEN), lambda i: (0, 0)),
            pl.BlockSpec((NUM_CLASSES, HIDDEN), lambda i: (0, 0)),
            pl.BlockSpec((1, NUM_CLASSES), lambda i: (0, 0)),
            pl.BlockSpec((NC, _MLP_ROWS, 8), lambda i: (0, i, 0)),
        ],
        out_specs=[
            pl.BlockSpec((_MLP_ROWS, NUM_CLASSES), lambda i: (i, 0)),
            pl.BlockSpec((_MLP_ROWS, NUM_CLASSES), lambda i: (i, 0)),
            pl.BlockSpec((_MLP_ROWS, NUM_CLASSES), lambda i: (i, 0)),
        ],
        out_shape=[
            jax.ShapeDtypeStruct((N, NUM_CLASSES), jnp.float32),
            jax.ShapeDtypeStruct((N, NUM_CLASSES), jnp.float32),
            jax.ShapeDtypeStruct((N, NUM_CLASSES), jnp.float32),
        ],
    )(feature, W1, b1.reshape(1, HIDDEN), W2, b2.reshape(1, NUM_CLASSES),
      degpart)


_CMB_ROWS = 2000


def _comb_body(scale, zp_ref, txp_ref, dis_ref, tx_ref, y_ref):
    z = zp_ref[0] + zp_ref[1]
    dis = dis_ref[...]
    tx = (-scale) * dis * z - txp_ref[...]
    tx_ref[...] = tx
    y_ref[...] = dis * tx


def _combine(zpart, tx_prev, dis16, scale):
    grid = N // _CMB_ROWS
    return pl.pallas_call(
        functools.partial(_comb_body, scale),
        grid=(grid,),
        in_specs=[
            pl.BlockSpec((NC, _CMB_ROWS, NUM_CLASSES), lambda i: (0, i, 0)),
            pl.BlockSpec((_CMB_ROWS, NUM_CLASSES), lambda i: (i, 0)),
            pl.BlockSpec((_CMB_ROWS, NUM_CLASSES), lambda i: (i, 0)),
        ],
        out_specs=[
            pl.BlockSpec((_CMB_ROWS, NUM_CLASSES), lambda i: (i, 0)),
            pl.BlockSpec((_CMB_ROWS, NUM_CLASSES), lambda i: (i, 0)),
        ],
        out_shape=[
            jax.ShapeDtypeStruct((N, NUM_CLASSES), jnp.float32),
            jax.ShapeDtypeStruct((N, NUM_CLASSES), jnp.float32),
        ],
    )(zpart, tx_prev, dis16)


_ETA_ROWS = 2000


def _eta_body(tx_ref, pw_ref, pb_ref, g_ref, eta_ref):
    cols = []
    for k in range(K + 1):
        h = lax.dot_general(tx_ref[k], pw_ref[k], (((1,), (1,)), ((), ())),
                            preferred_element_type=jnp.float32)
        h = jnp.tanh(h + pb_ref[0, k][None, :])
        col = lax.dot_general(h, g_ref[0, :, k:k + 1], (((1,), (0,)), ((), ())),
                              preferred_element_type=jnp.float32)
        cols.append(col / RANK)
    cols.append(jnp.zeros((_ETA_ROWS, NUM_CLASSES - (K + 1)), jnp.float32))
    eta_ref[...] = jnp.concatenate(cols, axis=1)


def _eta(tx_stack, PW, Pb, gamma):
    grid = N // _ETA_ROWS
    return pl.pallas_call(
        _eta_body,
        grid=(grid,),
        in_specs=[
            pl.BlockSpec((K + 1, _ETA_ROWS, NUM_CLASSES), lambda i: (0, i, 0)),
            pl.BlockSpec((K + 1, RANK, NUM_CLASSES), lambda i: (0, 0, 0)),
            pl.BlockSpec((1, K + 1, RANK), lambda i: (0, 0, 0)),
            pl.BlockSpec((1, RANK, K + 1), lambda i: (0, 0, 0)),
        ],
        out_specs=pl.BlockSpec((_ETA_ROWS, NUM_CLASSES), lambda i: (i, 0)),
        out_shape=jax.ShapeDtypeStruct((N, NUM_CLASSES), jnp.float32),
    )(tx_stack, PW, Pb.reshape(1, K + 1, RANK), gamma.reshape(1, RANK, K + 1))


_CTC_ROWS = 400


def _ctc_body(ctc_ref, eta_ref, tx_ref, o_ref):
    etaout = jnp.dot(ctc_ref[...], eta_ref[...],
                     preferred_element_type=jnp.float32)
    hidden = tx_ref[0] * etaout[:, 0:1]
    for k in range(1, K + 1):
        hidden = hidden + tx_ref[k] * etaout[:, k:k + 1]
    m = jnp.max(hidden, axis=1, keepdims=True)
    s = hidden - m
    lse = jnp.log(jnp.sum(jnp.exp(s), axis=1, keepdims=True))
    o_ref[...] = s - lse


def _ctc_combine(CTC, eta, tx_stack):
    grid = N // _CTC_ROWS
    return pl.pallas_call(
        _ctc_body,
        grid=(grid,),
        in_specs=[
            pl.BlockSpec((_CTC_ROWS, N), lambda i: (i, 0)),
            pl.BlockSpec((N, NUM_CLASSES), lambda i: (0, 0)),
            pl.BlockSpec((K + 1, _CTC_ROWS, NUM_CLASSES), lambda i: (0, i, 0)),
        ],
        out_specs=pl.BlockSpec((_CTC_ROWS, NUM_CLASSES), lambda i: (i, 0)),
        out_shape=jax.ShapeDtypeStruct((N, NUM_CLASSES), jnp.float32),
    )(CTC, eta, tx_stack)


def kernel(feature, edges, CTC, W1, b1, W2, b2, gamma, PW, Pb):
    src = edges[0]
    dst = edges[1]
    mask = src != dst
    dstp = jnp.where(mask, dst, N)
    srcd = jnp.where(mask, src, N)
    pad = EPAD - E
    srcg_p = jnp.concatenate(
        [src, jnp.zeros((pad,), jnp.int32)]).reshape(NW, CH, CHUNK)
    dstp_p = jnp.concatenate(
        [dstp, jnp.full((pad,), N, jnp.int32)]).reshape(NW, CH, CHUNK)
    srcd_p = jnp.concatenate(
        [srcd, jnp.full((pad,), N, jnp.int32)]).reshape(NW, CH, CHUNK)
    zinit = jnp.zeros((ZROWS, NUM_CLASSES), jnp.float32)
    zinit1 = jnp.zeros((ZROWS, 8), jnp.float32)
    ones1 = jnp.ones((CHUNK, 8), jnp.float32)

    degpart = _deg_sc(srcd_p, ones1, zinit1)
    x, dis16, y = _mlp(feature, W1, b1, W2, b2, degpart)

    z = _prop_sc(y, srcg_p, dstp_p, zinit)
    tx1, y = _combine(z, jnp.zeros((N, NUM_CLASSES), jnp.float32), dis16, 1.0)
    txs = [x, tx1]
    tx0 = x
    for _ in range(1, K):
        z = _prop_sc(y, srcg_p, dstp_p, zinit)
        tx2, y = _combine(z, tx0, dis16, 2.0)
        tx0 = txs[-1]
        txs.append(tx2)
    tx_stack = jnp.stack(txs)

    eta = _eta(tx_stack, PW, Pb, gamma)
    return _ctc_combine(CTC, eta, tx_stack)


# speedup vs baseline: 1.0369x; 1.0369x over previous
import functools

import jax
import jax.numpy as jnp
from jax import lax
from jax.experimental import pallas as pl
from jax.experimental.pallas import tpu as pltpu
from jax.experimental.pallas import tpu_sc as plsc

N = 10000
E = 320000
NUM_FEATURES = 128
HIDDEN = 64
NUM_CLASSES = 16
K = 10
RANK = 3

NC = 2
NS = 16
NW = NC * NS
CHUNK = 128
UNROLL = 16
CH = 80
EW = CH * CHUNK
EPAD = EW * NW
NPAD = 10112
ZROWS = NPAD // NS
OROWS = ZROWS


def _deg_sc_body(srcd_hbm, ones_hbm, zinit_hbm, degpart_hbm, zsh, didx, ones_v,
                 *ssems):
    c = lax.axis_index("c")
    s = lax.axis_index("s")
    wid = s * NC + c
    pltpu.sync_copy(zinit_hbm, zsh.at[pl.ds(s * ZROWS, ZROWS)])
    pltpu.sync_copy(srcd_hbm.at[wid], didx)
    pltpu.sync_copy(ones_hbm, ones_v)
    plsc.subcore_barrier()

    def body(g, carry):
        j = g * UNROLL
        for u in range(UNROLL):
            pltpu.async_copy(ones_v, zsh.at[didx.at[j + u]], ssems[u],
                             add=True)
        for u in range(UNROLL):
            pltpu.make_async_copy(ones_hbm, ones_v, ssems[u]).wait()
        return carry

    lax.fori_loop(0, CH // UNROLL, body, 0)
    plsc.subcore_barrier()
    pltpu.sync_copy(zsh.at[pl.ds(s * OROWS, OROWS)],
                    degpart_hbm.at[c, pl.ds(s * OROWS, OROWS)])


def _deg_sc(srcd_p, ones_v, zinit):
    mesh = plsc.VectorSubcoreMesh(core_axis_name="c", subcore_axis_name="s")
    fn = pl.kernel(
        _deg_sc_body,
        out_type=jax.ShapeDtypeStruct((NC, NPAD, 8), jnp.float32),
        mesh=mesh,
        scratch_types=(
            [pltpu.VMEM_SHARED((NPAD, 8), jnp.float32),
             pltpu.VMEM((CH, CHUNK), jnp.int32),
             pltpu.VMEM((CHUNK, 8), jnp.float32)]
            + [pltpu.SemaphoreType.DMA] * UNROLL
        ),
        compiler_params=pltpu.CompilerParams(use_tc_tiling_on_sc=False),
    )
    return fn(srcd_p, ones_v, zinit)


def _prop_sc_body(y_hbm, src_hbm, dst_hbm, zinit_hbm, zpart_hbm,
                  zsh, sidx, didx, *bufsems):
    c = lax.axis_index("c")
    s = lax.axis_index("s")
    wid = s * NC + c
    pltpu.sync_copy(zinit_hbm, zsh.at[pl.ds(s * ZROWS, ZROWS)])
    pltpu.sync_copy(src_hbm.at[wid], sidx)
    pltpu.sync_copy(dst_hbm.at[wid], didx)
    plsc.subcore_barrier()

    bufs = bufsems[:UNROLL]
    gsems = bufsems[UNROLL:]

    def group(g, carry):
        j = g * UNROLL
        gds = [pltpu.async_copy(y_hbm.at[sidx.at[j + u]], bufs[u], gsems[u])
               for u in range(UNROLL)]
        for u in range(UNROLL):
            gds[u].wait()
            pltpu.sync_copy(bufs[u], zsh.at[didx.at[j + u]], add=True)
        return carry

    lax.fori_loop(0, CH // UNROLL, group, 0)
    plsc.subcore_barrier()
    pltpu.sync_copy(zsh.at[pl.ds(s * OROWS, OROWS)],
                    zpart_hbm.at[c, pl.ds(s * OROWS, OROWS)])


def _prop_sc(y, srcg_p, dstp_p, zinit):
    mesh = plsc.VectorSubcoreMesh(core_axis_name="c", subcore_axis_name="s")
    fn = pl.kernel(
        _prop_sc_body,
        out_type=jax.ShapeDtypeStruct((NC, NPAD, NUM_CLASSES), jnp.float32),
        mesh=mesh,
        scratch_types=(
            [pltpu.VMEM_SHARED((NPAD, NUM_CLASSES), jnp.float32),
             pltpu.VMEM((CH, CHUNK), jnp.int32),
             pltpu.VMEM((CH, CHUNK), jnp.int32)]
            + [pltpu.VMEM((CHUNK, NUM_CLASSES), jnp.float32)] * UNROLL
            + [pltpu.SemaphoreType.DMA] * UNROLL
        ),
        compiler_params=pltpu.CompilerParams(use_tc_tiling_on_sc=False),
    )
    return fn(y, srcg_p, dstp_p, zinit)


_MLP_ROWS = 2000


def _mlp_body(f_ref, w1_ref, b1_ref, w2_ref, b2_ref, dp_ref,
              x_ref, dis_ref, y_ref):
    h = lax.dot_general(f_ref[...], w1_ref[...], (((1,), (1,)), ((), ())),
                        preferred_element_type=jnp.float32)
    h = jnp.maximum(h + b1_ref[...], 0.0)
    x = lax.dot_general(h, w2_ref[...], (((1,), (1,)), ((), ())),
                        preferred_element_type=jnp.float32)
    x = x + b2_ref[...]
    x_ref[...] = x
    deg = (dp_ref[0] + dp_ref[1])[:, 0:1]
    dis = jnp.where(deg > 0.0, lax.rsqrt(jnp.maximum(deg, 1e-30)), 0.0)
    disb = jnp.broadcast_to(dis, (dis.shape[0], NUM_CLASSES))
    dis_ref[...] = disb
    y_ref[...] = disb * x


def _mlp(feature, W1, b1, W2, b2, degpart):
    grid = N // _MLP_ROWS
    return pl.pallas_call(
        _mlp_body,
        grid=(grid,),
        in_specs=[
            pl.BlockSpec((_MLP_ROWS, NUM_FEATURES), lambda i: (i, 0)),
            pl.BlockSpec((HIDDEN, NUM_FEATURES), lambda i: (0, 0)),
            pl.BlockSpec((1, HIDDEN), lambda i: (0, 0)),
            pl.BlockSpec((NUM_CLASSES, HIDDEN), lambda i: (0, 0)),
            pl.BlockSpec((1, NUM_CLASSES), lambda i: (0, 0)),
            pl.BlockSpec((NC, _MLP_ROWS, 8), lambda i: (0, i, 0)),
        ],
        out_specs=[
            pl.BlockSpec((_MLP_ROWS, NUM_CLASSES), lambda i: (i, 0)),
            pl.BlockSpec((_MLP_ROWS, NUM_CLASSES), lambda i: (i, 0)),
            pl.BlockSpec((_MLP_ROWS, NUM_CLASSES), lambda i: (i, 0)),
        ],
        out_shape=[
            jax.ShapeDtypeStruct((N, NUM_CLASSES), jnp.float32),
            jax.ShapeDtypeStruct((N, NUM_CLASSES), jnp.float32),
            jax.ShapeDtypeStruct((N, NUM_CLASSES), jnp.float32),
        ],
    )(feature, W1, b1.reshape(1, HIDDEN), W2, b2.reshape(1, NUM_CLASSES),
      degpart)


_CMB_ROWS = 2000


def _comb_body(scale, zp_ref, txp_ref, dis_ref, tx_ref, y_ref):
    z = zp_ref[0] + zp_ref[1]
    dis = dis_ref[...]
    tx = (-scale) * dis * z - txp_ref[...]
    tx_ref[...] = tx
    y_ref[...] = dis * tx


def _combine(zpart, tx_prev, dis16, scale):
    grid = N // _CMB_ROWS
    return pl.pallas_call(
        functools.partial(_comb_body, scale),
        grid=(grid,),
        in_specs=[
            pl.BlockSpec((NC, _CMB_ROWS, NUM_CLASSES), lambda i: (0, i, 0)),
            pl.BlockSpec((_CMB_ROWS, NUM_CLASSES), lambda i: (i, 0)),
            pl.BlockSpec((_CMB_ROWS, NUM_CLASSES), lambda i: (i, 0)),
        ],
        out_specs=[
            pl.BlockSpec((_CMB_ROWS, NUM_CLASSES), lambda i: (i, 0)),
            pl.BlockSpec((_CMB_ROWS, NUM_CLASSES), lambda i: (i, 0)),
        ],
        out_shape=[
            jax.ShapeDtypeStruct((N, NUM_CLASSES), jnp.float32),
            jax.ShapeDtypeStruct((N, NUM_CLASSES), jnp.float32),
        ],
    )(zpart, tx_prev, dis16)


_ETA_ROWS = 2000


def _eta_body(tx_ref, pw_ref, pb_ref, g_ref, eta_ref):
    cols = []
    for k in range(K + 1):
        h = lax.dot_general(tx_ref[k], pw_ref[k], (((1,), (1,)), ((), ())),
                            preferred_element_type=jnp.float32)
        h = jnp.tanh(h + pb_ref[0, k][None, :])
        col = lax.dot_general(h, g_ref[0, :, k:k + 1], (((1,), (0,)), ((), ())),
                              preferred_element_type=jnp.float32)
        cols.append(col / RANK)
    cols.append(jnp.zeros((_ETA_ROWS, NUM_CLASSES - (K + 1)), jnp.float32))
    eta_ref[...] = jnp.concatenate(cols, axis=1)


def _eta(tx_stack, PW, Pb, gamma):
    grid = N // _ETA_ROWS
    return pl.pallas_call(
        _eta_body,
        grid=(grid,),
        in_specs=[
            pl.BlockSpec((K + 1, _ETA_ROWS, NUM_CLASSES), lambda i: (0, i, 0)),
            pl.BlockSpec((K + 1, RANK, NUM_CLASSES), lambda i: (0, 0, 0)),
            pl.BlockSpec((1, K + 1, RANK), lambda i: (0, 0, 0)),
            pl.BlockSpec((1, RANK, K + 1), lambda i: (0, 0, 0)),
        ],
        out_specs=pl.BlockSpec((_ETA_ROWS, NUM_CLASSES), lambda i: (i, 0)),
        out_shape=jax.ShapeDtypeStruct((N, NUM_CLASSES), jnp.float32),
    )(tx_stack, PW, Pb.reshape(1, K + 1, RANK), gamma.reshape(1, RANK, K + 1))


_CTC_ROWS = 400


def _ctc_body(ctc_ref, eta_ref, tx_ref, o_ref):
    etaout = jnp.dot(ctc_ref[...], eta_ref[...],
                     preferred_element_type=jnp.float32)
    hidden = tx_ref[0] * etaout[:, 0:1]
    for k in range(1, K + 1):
        hidden = hidden + tx_ref[k] * etaout[:, k:k + 1]
    m = jnp.max(hidden, axis=1, keepdims=True)
    s = hidden - m
    lse = jnp.log(jnp.sum(jnp.exp(s), axis=1, keepdims=True))
    o_ref[...] = s - lse


def _ctc_combine(CTC, eta, tx_stack):
    grid = N // _CTC_ROWS
    return pl.pallas_call(
        _ctc_body,
        grid=(grid,),
        in_specs=[
            pl.BlockSpec((_CTC_ROWS, N), lambda i: (i, 0)),
            pl.BlockSpec((N, NUM_CLASSES), lambda i: (0, 0)),
            pl.BlockSpec((K + 1, _CTC_ROWS, NUM_CLASSES), lambda i: (0, i, 0)),
        ],
        out_specs=pl.BlockSpec((_CTC_ROWS, NUM_CLASSES), lambda i: (i, 0)),
        out_shape=jax.ShapeDtypeStruct((N, NUM_CLASSES), jnp.float32),
    )(CTC, eta, tx_stack)


def kernel(feature, edges, CTC, W1, b1, W2, b2, gamma, PW, Pb):
    src = edges[0]
    dst = edges[1]
    mask = src != dst
    dstp = jnp.where(mask, dst, N)
    srcd = jnp.where(mask, src, N)
    pad = EPAD - E
    srcg_p = jnp.concatenate(
        [src, jnp.zeros((pad,), jnp.int32)]).reshape(NW, CH, CHUNK)
    dstp_p = jnp.concatenate(
        [dstp, jnp.full((pad,), N, jnp.int32)]).reshape(NW, CH, CHUNK)
    srcd_p = jnp.concatenate(
        [srcd, jnp.full((pad,), N, jnp.int32)]).reshape(NW, CH, CHUNK)
    zinit = jnp.zeros((ZROWS, NUM_CLASSES), jnp.float32)
    zinit1 = jnp.zeros((ZROWS, 8), jnp.float32)
    ones1 = jnp.ones((CHUNK, 8), jnp.float32)

    degpart = _deg_sc(srcd_p, ones1, zinit1)
    x, dis16, y = _mlp(feature, W1, b1, W2, b2, degpart)

    z = _prop_sc(y, srcg_p, dstp_p, zinit)
    tx1, y = _combine(z, jnp.zeros((N, NUM_CLASSES), jnp.float32), dis16, 1.0)
    txs = [x, tx1]
    tx0 = x
    for _ in range(1, K):
        z = _prop_sc(y, srcg_p, dstp_p, zinit)
        tx2, y = _combine(z, tx0, dis16, 2.0)
        tx0 = txs[-1]
        txs.append(tx2)
    tx_stack = jnp.stack(txs)

    eta = _eta(tx_stack, PW, Pb, gamma)
    return _ctc_combine(CTC, eta, tx_stack)
